# baseline (speedup 1.0000x reference)
import jax
import jax.numpy as jnp
from jax.experimental import pallas as pl
from jax.experimental.pallas import tpu as pltpu

_BM = 512


def _proj_kernel(b_ref, w_ref, bias_ref, h_ref):
    h = (
        jnp.dot(b_ref[...], w_ref[...], preferred_element_type=jnp.float32)
        + bias_ref[...]
    )
    h_ref[...] = h.astype(jnp.bfloat16)


def _stream_kernel(a_ref, h_ref, out_ref):
    a16 = a_ref[...].astype(jnp.bfloat16)
    out_ref[...] = jnp.dot(a16, h_ref[...], preferred_element_type=jnp.float32)


def kernel(a, b, W, bias):
    n, k = a.shape
    d_in, d_out = W.shape
    bias2 = bias.reshape(1, d_out)

    h = pl.pallas_call(
        _proj_kernel,
        in_specs=[
            pl.BlockSpec((k, d_in), lambda: (0, 0)),
            pl.BlockSpec((d_in, d_out), lambda: (0, 0)),
            pl.BlockSpec((1, d_out), lambda: (0, 0)),
        ],
        out_specs=pl.BlockSpec((k, d_out), lambda: (0, 0)),
        out_shape=jax.ShapeDtypeStruct((k, d_out), jnp.bfloat16),
    )(b, W, bias2)

    return pl.pallas_call(
        _stream_kernel,
        grid=(n // _BM,),
        in_specs=[
            pl.BlockSpec((_BM, k), lambda i: (i, 0)),
            pl.BlockSpec((k, d_out), lambda i: (0, 0)),
        ],
        out_specs=pl.BlockSpec((_BM, d_out), lambda i: (i, 0)),
        out_shape=jax.ShapeDtypeStruct((n, d_out), jnp.float32),
    )(a, h)


# speedup vs baseline: 1.0934x; 1.0934x over previous
import jax
import jax.numpy as jnp
from jax.experimental import pallas as pl
from jax.experimental.pallas import tpu as pltpu

_BM = 512


def _fused_kernel(a_ref, b_ref, w_ref, bias_ref, out_ref, h_ref):
    @pl.when(pl.program_id(0) == 0)
    def _():
        h = (
            jnp.dot(b_ref[...], w_ref[...], preferred_element_type=jnp.float32)
            + bias_ref[...]
        )
        h_ref[...] = h.astype(jnp.bfloat16)

    a16 = a_ref[...].astype(jnp.bfloat16)
    out_ref[...] = jnp.dot(a16, h_ref[...], preferred_element_type=jnp.float32)


def kernel(a, b, W, bias):
    n, k = a.shape
    d_in, d_out = W.shape
    bias2 = bias.reshape(1, d_out)
    return pl.pallas_call(
        _fused_kernel,
        grid=(n // _BM,),
        in_specs=[
            pl.BlockSpec((_BM, k), lambda i: (i, 0)),
            pl.BlockSpec((k, d_in), lambda i: (0, 0)),
            pl.BlockSpec((d_in, d_out), lambda i: (0, 0)),
            pl.BlockSpec((1, d_out), lambda i: (0, 0)),
        ],
        out_specs=pl.BlockSpec((_BM, d_out), lambda i: (i, 0)),
        out_shape=jax.ShapeDtypeStruct((n, d_out), jnp.float32),
        scratch_shapes=[pltpu.VMEM((k, d_out), jnp.bfloat16)],
    )(a, b, W, bias2)
